# Initial kernel scaffold; baseline (speedup 1.0000x reference)
import jax
import jax.numpy as jnp
from jax.experimental import pallas as pl


def kernel(x, edge_index, graph_ids, gin_W, gin_b, mlp_W1, mlp_b1, bn1_g, bn1_b, mlp_W2, mlp_b2, bn2_g, bn2_b, mlp_W3, mlp_b3):
    raise NotImplementedError("write your pallas kernel here")


# speedup vs baseline: 1.9216x; 1.9216x over previous
import jax
import jax.numpy as jnp
from jax import lax
from jax.experimental import pallas as pl
from jax.experimental.pallas import tpu as pltpu
from jax.experimental.pallas import tpu_sc as plsc

N = 10000
D = 128
G = 256
NC = 2
NS = 16
NW = NC * NS
NPAD = 10240
ECH = 128
GPAD = 272
RPT = NPAD // NW

_mesh = plsc.VectorSubcoreMesh(core_axis_name="c", subcore_axis_name="s")


def _fill(ref, nrows, value):
    vec = jnp.full((16,), value, jnp.float32)

    def body(r, carry):
        for k in range(D // 16):
            ref[r, pl.ds(k * 16, 16)] = vec
        return carry

    lax.fori_loop(0, nrows, body, 0)


def _segsum_body(h_hbm, ssrc_hbm, sdst_hbm, bnd_hbm, out_hbm,
                 idx_v, dst_v, rows_v, bnd_v, acc_v, sem):
    c = lax.axis_index("c")
    s = lax.axis_index("s")
    wid = c * NS + s
    base = wid * RPT

    _fill(acc_v, RPT, 0.0)
    pltpu.sync_copy(bnd_hbm, bnd_v)
    b_lo = bnd_v[pl.ds(wid, 16)][0]
    b_hi = bnd_v[pl.ds(wid + 1, 16)][0]
    k0 = b_lo // ECH
    k1 = jnp.maximum(k0, (b_hi + ECH - 1) // ECH)

    def chunk(k, carry):
        pltpu.sync_copy(ssrc_hbm.at[k], idx_v)
        pltpu.sync_copy(sdst_hbm.at[k], dst_v.at[pl.ds(0, ECH)])
        pltpu.async_copy(h_hbm.at[idx_v], rows_v, sem).wait()
        gbase = k * ECH

        def row(r, carry2):
            @pl.when(((gbase + r) >= b_lo) & ((gbase + r) < b_hi))
            def _():
                li = dst_v[pl.ds(r, 16)][0] - base
                for kk in range(D // 16):
                    sl = pl.ds(kk * 16, 16)
                    acc_v[li, sl] = acc_v[li, sl] + rows_v[r, sl]
            return carry2

        lax.fori_loop(0, ECH, row, 0)
        return carry

    lax.fori_loop(k0, k1, chunk, 0)
    pltpu.sync_copy(acc_v, out_hbm.at[pl.ds(base, RPT)])


def _segsum(h_pad, ssrc3, sdst3, bnd):
    f = pl.kernel(
        _segsum_body,
        out_type=jax.ShapeDtypeStruct((NPAD, D), jnp.float32),
        mesh=_mesh,
        scratch_types=[
            pltpu.VMEM((ECH,), jnp.int32),
            pltpu.VMEM((ECH + 16,), jnp.int32),
            pltpu.VMEM((ECH, D), jnp.float32),
            pltpu.VMEM((64,), jnp.int32),
            pltpu.VMEM((RPT, D), jnp.float32),
            pltpu.SemaphoreType.DMA,
        ],
    )
    return f(h_pad, ssrc3, sdst3, bnd)


def _pool_body(h_hbm, gid_hbm, sums_out, cnts_out,
               gid_v, rows_v, ones_v, zeros_v, pool_sh, cnt_sh, sem):
    c = lax.axis_index("c")
    s = lax.axis_index("s")
    wid = c * NS + s

    _fill(zeros_v, GPAD // NS, 0.0)
    _fill(ones_v, 64, 1.0)

    pltpu.sync_copy(zeros_v, pool_sh.at[pl.ds(s * (GPAD // NS), GPAD // NS)])
    pltpu.sync_copy(zeros_v, cnt_sh.at[pl.ds(s * (GPAD // NS), GPAD // NS)])
    pltpu.sync_copy(gid_hbm.at[wid], gid_v)
    plsc.subcore_barrier()

    def chunk(t, carry):
        base = wid * (NPAD // NW) + t * 64
        pltpu.sync_copy(h_hbm.at[pl.ds(base, 64)], rows_v)
        pltpu.sync_copy(rows_v, pool_sh.at[gid_v.at[t]], add=True)
        pltpu.sync_copy(ones_v, cnt_sh.at[gid_v.at[t]], add=True)
        return carry

    lax.fori_loop(0, (NPAD // NW) // 64, chunk, 0)
    plsc.subcore_barrier()

    pltpu.sync_copy(pool_sh.at[pl.ds(s * (G // NS), G // NS)],
                    sums_out.at[c, pl.ds(s * (G // NS), G // NS)])
    pltpu.sync_copy(cnt_sh.at[pl.ds(s * (G // NS), G // NS)],
                    cnts_out.at[c, pl.ds(s * (G // NS), G // NS)])


def _pool(h_pad, gids3):
    f = pl.kernel(
        _pool_body,
        out_type=[
            jax.ShapeDtypeStruct((NC, G, D), jnp.float32),
            jax.ShapeDtypeStruct((NC, G, D), jnp.float32),
        ],
        mesh=_mesh,
        scratch_types=[
            pltpu.VMEM(gids3.shape[1:], jnp.int32),
            pltpu.VMEM((64, D), jnp.float32),
            pltpu.VMEM((64, D), jnp.float32),
            pltpu.VMEM((GPAD // NS, D), jnp.float32),
            pltpu.VMEM_SHARED((GPAD, D), jnp.float32),
            pltpu.VMEM_SHARED((GPAD, D), jnp.float32),
            pltpu.SemaphoreType.DMA,
        ],
    )
    return f(h_pad, gids3)


def _gin_matmul_body(h_ref, a_ref, w_ref, b_ref, o_ref):
    t = h_ref[...] + a_ref[...]
    o_ref[...] = jnp.maximum(
        jnp.dot(t, w_ref[...], preferred_element_type=jnp.float32)
        + b_ref[...], 0.0)


def _gin_matmul(h_pad, agg, w, b):
    bm = 1024
    return pl.pallas_call(
        _gin_matmul_body,
        grid=(NPAD // bm,),
        in_specs=[
            pl.BlockSpec((bm, D), lambda i: (i, 0)),
            pl.BlockSpec((bm, D), lambda i: (i, 0)),
            pl.BlockSpec((D, D), lambda i: (0, 0)),
            pl.BlockSpec((1, D), lambda i: (0, 0)),
        ],
        out_specs=pl.BlockSpec((bm, D), lambda i: (i, 0)),
        out_shape=jax.ShapeDtypeStruct((NPAD, D), jnp.float32),
    )(h_pad, agg, w, b.reshape(1, D))


def _head_body(s0, s1, c0, c1, w1, b1, g1, bb1, w2, b2, g2, bb2, w3, b3, o):
    nm = (s0[...] + s1[...]) / jnp.maximum(c0[...] + c1[...], 1.0)
    z = jnp.dot(nm, w1[...], preferred_element_type=jnp.float32) + b1[...]
    mu = jnp.mean(z, axis=0, keepdims=True)
    var = jnp.mean((z - mu) ** 2, axis=0, keepdims=True)
    z = (z - mu) / jnp.sqrt(var + 1e-5) * g1[...] + bb1[...]
    z = jnp.where(z > 0, z, 0.01 * z)
    z = jnp.dot(z, w2[...], preferred_element_type=jnp.float32) + b2[...]
    mu = jnp.mean(z, axis=0, keepdims=True)
    var = jnp.mean((z - mu) ** 2, axis=0, keepdims=True)
    z = (z - mu) / jnp.sqrt(var + 1e-5) * g2[...] + bb2[...]
    z = jnp.where(z > 0, z, 0.01 * z)
    o[...] = jnp.sum(z * w3[...], axis=1, keepdims=True) + b3[...]


def _head(sums, cnts, w1, b1, g1, bb1, w2, b2, g2, bb2, w3, b3):
    return pl.pallas_call(
        _head_body,
        out_shape=jax.ShapeDtypeStruct((G, 1), jnp.float32),
    )(sums[0], sums[1], cnts[0], cnts[1],
      w1, b1.reshape(1, -1), g1.reshape(1, -1), bb1.reshape(1, -1),
      w2, b2.reshape(1, -1), g2.reshape(1, -1), bb2.reshape(1, -1),
      w3.reshape(1, D), b3.reshape(1, 1))


def kernel(x, edge_index, graph_ids, gin_W, gin_b,
           mlp_W1, mlp_b1, bn1_g, bn1_b,
           mlp_W2, mlp_b2, bn2_g, bn2_b,
           mlp_W3, mlp_b3):
    e = edge_index.shape[1]
    epad = ECH * (-(-e // ECH))

    x_pad = jnp.pad(x, ((0, NPAD - N), (0, 0)))

    perm = jnp.argsort(edge_index[1], stable=True)
    ssrc = jnp.pad(jnp.take(edge_index[0], perm), (0, epad - e))
    sdst = jnp.pad(jnp.take(edge_index[1], perm), (0, epad - e),
                   constant_values=NPAD - 1)
    bnd = jnp.searchsorted(sdst, jnp.arange(0, NPAD + 1, RPT)).astype(jnp.int32)
    bnd = jnp.pad(bnd, (0, 64 - bnd.shape[0]))
    ssrc3 = ssrc.reshape(epad // ECH, ECH)
    sdst3 = sdst.reshape(epad // ECH, ECH)

    gids3 = jnp.pad(graph_ids, (0, NPAD - N),
                    constant_values=G).reshape(NW, (NPAD // NW) // 64, 64)

    h = x_pad
    for i in range(gin_W.shape[0]):
        agg = _segsum(h, ssrc3, sdst3, bnd)
        h = _gin_matmul(h, agg, gin_W[i], gin_b[i])

    sums, cnts = _pool(h, gids3)
    out = _head(sums, cnts, mlp_W1, mlp_b1, bn1_g, bn1_b,
                mlp_W2, mlp_b2, bn2_g, bn2_b, mlp_W3, mlp_b3)
    return jnp.squeeze(out, axis=-1)
